# baseline (speedup 1.0000x reference)
import dataclasses
import functools

import jax
import jax.numpy as jnp
from jax import lax
from jax.experimental import pallas as pl
from jax.experimental.pallas import tpu as pltpu
from jax.experimental.pallas import tpu_sc as plsc

_CH = 1024


def _sc_hardmax(n, m, n_workers):
    nch = n // _CH
    cw = _CH * m
    bw = n * m
    mesh = plsc.VectorSubcoreMesh(core_axis_name="c", subcore_axis_name="s",
                                  num_cores=2, num_subcores=16)
    cp = pltpu.CompilerParams()
    if "needs_layout_passes" in pltpu.CompilerParams.__dataclass_fields__:
        cp = dataclasses.replace(cp, needs_layout_passes=False)
    if "use_tc_tiling_on_sc" in pltpu.CompilerParams.__dataclass_fields__:
        cp = dataclasses.replace(cp, use_tc_tiling_on_sc=False)

    @functools.partial(
        pl.kernel,
        compiler_params=cp,
        out_type=jax.ShapeDtypeStruct((n_workers * bw,), jnp.float32),
        mesh=mesh,
        scratch_types=[
            pltpu.VMEM((cw,), jnp.float32),
            pltpu.VMEM((cw,), jnp.float32),
            pltpu.VMEM((cw,), jnp.float32),
            pltpu.VMEM((16,), jnp.float32),
            pltpu.VMEM((16,), jnp.float32),
            pltpu.VMEM((16,), jnp.int32),
            pltpu.SemaphoreType.DMA,
            pltpu.SemaphoreType.DMA,
            pltpu.SemaphoreType.DMA,
        ],
    )
    def k(x_hbm, o_hbm, buf0, buf1, zbuf, ones, gmax, gidx,
          rsem, zsem, ssem):
        big = jnp.int32(1 << 30)
        wid = lax.axis_index("s") * 2 + lax.axis_index("c")
        base = wid * bw

        zrow = jnp.zeros((16,), jnp.float32)

        @pl.loop(0, _CH)
        def _(r):
            zbuf[pl.ds(r * 16, 16)] = zrow

        ones[...] = jnp.full((16,), 1.0, jnp.float32)
        gmax[...] = jnp.full((16,), -jnp.inf, jnp.float32)
        gidx[...] = jnp.zeros((16,), jnp.int32)

        def process(buf, c):
            chunk_base = c * _CH

            neg = jnp.full((16,), -jnp.inf, jnp.float32)

            def body_a(t, accs):
                r = t * 128
                return tuple(
                    jnp.maximum(accs[i], buf[pl.ds(r + i * 16, 16)])
                    for i in range(8))

            accs = lax.fori_loop(0, _CH // 8, body_a, (neg,) * 8)
            cmx = accs[0]
            for i in range(1, 8):
                cmx = jnp.maximum(cmx, accs[i])

            gm = gmax[...]
            better = cmx > gm

            @pl.when(jnp.any(better))
            def _():
                def body_b(t, bidx):
                    eqm = buf[pl.ds(t * 16, 16)] == cmx
                    rfull = jnp.full((16,), t, jnp.int32)
                    return jnp.minimum(bidx, jnp.where(eqm, rfull, big))

                bidx = lax.fori_loop(0, _CH, body_b,
                                     jnp.full((16,), big, jnp.int32))
                gmax[...] = jnp.where(better, cmx, gm)
                gidx[...] = jnp.where(better, bidx + chunk_base, gidx[...])

        pltpu.async_copy(x_hbm.at[pl.ds(base, cw)], buf0, rsem)
        pltpu.async_copy(x_hbm.at[pl.ds(base + cw, cw)], buf1, rsem)

        @pl.loop(0, nch // 2)
        def _(g):
            c0 = g * 2
            pltpu.async_copy(
                zbuf, o_hbm.at[pl.ds(base + c0 * cw, cw)], zsem)
            pltpu.async_copy(
                zbuf, o_hbm.at[pl.ds(base + (c0 + 1) * cw, cw)], zsem)
            pltpu.make_async_copy(
                x_hbm.at[pl.ds(base, cw)], buf0, rsem).wait()
            process(buf0, c0)

            @pl.when(g < nch // 2 - 1)
            def _():
                pltpu.async_copy(
                    x_hbm.at[pl.ds(base + (c0 + 2) * cw, cw)], buf0, rsem)

            pltpu.make_async_copy(
                x_hbm.at[pl.ds(base, cw)], buf1, rsem).wait()
            process(buf1, c0 + 1)

            @pl.when(g < nch // 2 - 1)
            def _():
                pltpu.async_copy(
                    x_hbm.at[pl.ds(base + (c0 + 3) * cw, cw)], buf1, rsem)

        wvec = (gidx[...] + wid * n) * m + lax.iota(jnp.int32, 16)

        @pl.loop(0, nch)
        def _(c):
            pltpu.make_async_copy(
                zbuf, o_hbm.at[pl.ds(base, cw)], zsem).wait()

        pltpu.async_copy(ones, o_hbm.at[wvec], ssem).wait()

    return k


def kernel(x):
    b, n, m = x.shape
    out = _sc_hardmax(n, m, b)(x.reshape(b * n * m))
    return out.reshape(b, n, m)


# speedup vs baseline: 1.0402x; 1.0402x over previous
import dataclasses
import functools

import jax
import jax.numpy as jnp
from jax import lax
from jax.experimental import pallas as pl
from jax.experimental.pallas import tpu as pltpu
from jax.experimental.pallas import tpu_sc as plsc

_CH = 256


def _sc_hardmax(n, m, n_workers):
    nch = n // _CH
    cw = _CH * m
    bw = n * m
    mesh = plsc.VectorSubcoreMesh(core_axis_name="c", subcore_axis_name="s",
                                  num_cores=2, num_subcores=16)
    cp = pltpu.CompilerParams()
    if "needs_layout_passes" in pltpu.CompilerParams.__dataclass_fields__:
        cp = dataclasses.replace(cp, needs_layout_passes=False)
    if "use_tc_tiling_on_sc" in pltpu.CompilerParams.__dataclass_fields__:
        cp = dataclasses.replace(cp, use_tc_tiling_on_sc=False)

    @functools.partial(
        pl.kernel,
        compiler_params=cp,
        out_type=jax.ShapeDtypeStruct((n_workers * bw,), jnp.float32),
        mesh=mesh,
        scratch_types=[
            pltpu.VMEM((cw,), jnp.float32),
            pltpu.VMEM((cw,), jnp.float32),
            pltpu.VMEM((cw,), jnp.float32),
            pltpu.VMEM((16,), jnp.float32),
            pltpu.VMEM((16,), jnp.float32),
            pltpu.VMEM((16,), jnp.int32),
            pltpu.SemaphoreType.DMA,
            pltpu.SemaphoreType.DMA,
            pltpu.SemaphoreType.DMA,
        ],
    )
    def k(x_hbm, o_hbm, buf0, buf1, zbuf, ones, gmax, gidx,
          rsem, zsem, ssem):
        big = jnp.int32(1 << 30)
        wid = lax.axis_index("s") * 2 + lax.axis_index("c")
        base = wid * bw

        zrow = jnp.zeros((16,), jnp.float32)

        @pl.loop(0, _CH)
        def _(r):
            zbuf[pl.ds(r * 16, 16)] = zrow

        ones[...] = jnp.full((16,), 1.0, jnp.float32)
        gmax[...] = jnp.full((16,), -jnp.inf, jnp.float32)
        gidx[...] = jnp.zeros((16,), jnp.int32)

        def process(buf, c):
            chunk_base = c * _CH

            neg = jnp.full((16,), -jnp.inf, jnp.float32)

            def body_a(t, accs):
                r = t * 128
                return tuple(
                    jnp.maximum(accs[i], buf[pl.ds(r + i * 16, 16)])
                    for i in range(8))

            accs = lax.fori_loop(0, _CH // 8, body_a, (neg,) * 8)
            cmx = accs[0]
            for i in range(1, 8):
                cmx = jnp.maximum(cmx, accs[i])

            gm = gmax[...]
            better = cmx > gm

            @pl.when(jnp.any(better))
            def _():
                def body_b(t, bidx):
                    eqm = buf[pl.ds(t * 16, 16)] == cmx
                    rfull = jnp.full((16,), t, jnp.int32)
                    return jnp.minimum(bidx, jnp.where(eqm, rfull, big))

                bidx = lax.fori_loop(0, _CH, body_b,
                                     jnp.full((16,), big, jnp.int32))
                gmax[...] = jnp.where(better, cmx, gm)
                gidx[...] = jnp.where(better, bidx + chunk_base, gidx[...])

        pltpu.async_copy(x_hbm.at[pl.ds(base, cw)], buf0, rsem)
        pltpu.async_copy(x_hbm.at[pl.ds(base + cw, cw)], buf1, rsem)

        @pl.loop(0, nch // 2)
        def _(g):
            c0 = g * 2
            pltpu.async_copy(
                zbuf, o_hbm.at[pl.ds(base + c0 * cw, cw)], zsem)
            pltpu.async_copy(
                zbuf, o_hbm.at[pl.ds(base + (c0 + 1) * cw, cw)], zsem)
            pltpu.make_async_copy(
                x_hbm.at[pl.ds(base, cw)], buf0, rsem).wait()
            process(buf0, c0)

            @pl.when(g < nch // 2 - 1)
            def _():
                pltpu.async_copy(
                    x_hbm.at[pl.ds(base + (c0 + 2) * cw, cw)], buf0, rsem)

            pltpu.make_async_copy(
                x_hbm.at[pl.ds(base, cw)], buf1, rsem).wait()
            process(buf1, c0 + 1)

            @pl.when(g < nch // 2 - 1)
            def _():
                pltpu.async_copy(
                    x_hbm.at[pl.ds(base + (c0 + 3) * cw, cw)], buf1, rsem)

        wvec = (gidx[...] + wid * n) * m + lax.iota(jnp.int32, 16)

        @pl.loop(0, nch)
        def _(c):
            pltpu.make_async_copy(
                zbuf, o_hbm.at[pl.ds(base, cw)], zsem).wait()

        pltpu.async_copy(ones, o_hbm.at[wvec], ssem).wait()

    return k


def kernel(x):
    b, n, m = x.shape
    out = _sc_hardmax(n, m, b)(x.reshape(b * n * m))
    return out.reshape(b, n, m)
